# baseline (speedup 1.0000x reference)
import functools

import jax
import jax.numpy as jnp
from jax import lax
from jax.experimental import pallas as pl
from jax.experimental.pallas import tpu as pltpu
from jax.experimental.pallas import tpu_sc as plsc

_VOCAB = 30522
_NENT = 100000
_EDIM = 100
_HID = 768
_MAXP = 512
_B = 16
_S = 256
_NTOK = _B * _S
_NW = 32
_TPW = _NTOK // _NW
_EPS = 1e-12
_PT = _MAXP + 2


def _splat_last(x, L=16):
    idx = jnp.full((L, 1), L - 1, jnp.int32)
    dn = lax.GatherDimensionNumbers(offset_dims=(), collapsed_slice_dims=(0,),
                                    start_index_map=(0,))
    return lax.gather(x, idx, dn, (1,),
                      mode=lax.GatherScatterMode.PROMISE_IN_BOUNDS)


def _sc_split(ids_f, tts_f):
    mesh = plsc.VectorSubcoreMesh(core_axis_name="c", subcore_axis_name="s")
    L = 16
    nchunk = _S // L

    @functools.partial(
        pl.kernel,
        mesh=mesh,
        compiler_params=pltpu.CompilerParams(needs_layout_passes=False),
        out_type=[jax.ShapeDtypeStruct((_NTOK,), jnp.int32)] * 6
                 + [jax.ShapeDtypeStruct((_NTOK,), jnp.float32)],
        scratch_types=[
            pltpu.VMEM((_S,), jnp.int32),
            pltpu.VMEM((_S,), jnp.int32),
            pltpu.VMEM((_S,), jnp.int32),
            pltpu.VMEM((_S,), jnp.int32),
            pltpu.VMEM((_S,), jnp.int32),
            pltpu.VMEM((_S,), jnp.int32),
            pltpu.VMEM((_S,), jnp.int32),
            pltpu.VMEM((_S,), jnp.int32),
            pltpu.VMEM((_S,), jnp.float32),
        ],
    )
    def split(ids_h, tts_h, wid_o, wtt_o, wpos_o, kent_o, ktt_o, kpos_o, kmsk_o,
              idsv, ttsv, wiv, wtv, wpv, kev, ktv, kpv, kmv):
        wid = lax.axis_index("s") * 2 + lax.axis_index("c")

        @pl.when(wid < _B)
        def _():
            base = wid * _S
            pltpu.sync_copy(ids_h.at[pl.ds(base, _S)], idsv)
            pltpu.sync_copy(tts_h.at[pl.ds(base, _S)], ttsv)
            nwv = jnp.zeros((L,), jnp.int32)
            nkv = jnp.zeros((L,), jnp.int32)
            for c in range(nchunk):
                v = idsv[pl.ds(c * L, L)]
                t = ttsv[pl.ds(c * L, L)]
                colv = lax.iota(jnp.int32, L) + (c * L)
                wm = (v > 0) & (v < _VOCAB)
                wmi = jnp.where(wm, jnp.int32(1), jnp.int32(0))
                wcum = plsc.cumsum(wmi)
                wdest = jnp.where(wm, nwv + wcum - 1, 0)
                plsc.store_scatter(wiv, [wdest], v, mask=wm)
                plsc.store_scatter(wtv, [wdest], t, mask=wm)
                plsc.store_scatter(wpv, [wdest], colv, mask=wm)
                nwv = nwv + _splat_last(wcum)
                km = v >= _VOCAB
                kmi = jnp.where(km, jnp.int32(1), jnp.int32(0))
                kcum = plsc.cumsum(kmi)
                kdest = jnp.where(km, nkv + kcum - 1, 0)
                plsc.store_scatter(kev, [kdest], v - _VOCAB, mask=km)
                plsc.store_scatter(ktv, [kdest], t, mask=km)
                plsc.store_scatter(kpv, [kdest], colv, mask=km)
                nkv = nkv + _splat_last(kcum)
            nkeff = jnp.where(nkv >= 2, nkv, 0)
            for c in range(nchunk):
                sl = pl.ds(c * L, L)
                colv = lax.iota(jnp.int32, L) + (c * L)
                wvld = colv < nwv
                wiv[sl] = jnp.where(wvld, wiv[sl], 0)
                wtv[sl] = jnp.where(wvld, wtv[sl], 1)
                wpv[sl] = jnp.where(wvld, wpv[sl], colv)
                kvld = colv < nkeff
                kev[sl] = jnp.where(kvld, kev[sl], 0)
                ktv[sl] = jnp.where(kvld, ktv[sl], 0)
                kpv[sl] = jnp.where(kvld, kpv[sl], 0)
                kmv[sl] = jnp.where(kvld, 1.0, 0.0)
            pltpu.sync_copy(wiv, wid_o.at[pl.ds(base, _S)])
            pltpu.sync_copy(wtv, wtt_o.at[pl.ds(base, _S)])
            pltpu.sync_copy(wpv, wpos_o.at[pl.ds(base, _S)])
            pltpu.sync_copy(kev, kent_o.at[pl.ds(base, _S)])
            pltpu.sync_copy(ktv, ktt_o.at[pl.ds(base, _S)])
            pltpu.sync_copy(kpv, kpos_o.at[pl.ds(base, _S)])
            pltpu.sync_copy(kmv, kmsk_o.at[pl.ds(base, _S)])

    return split(ids_f, tts_f)


_EBLK = 2048


def _relayout_body(entT, eye, out):
    x = entT[...]
    xt = lax.dot_general(x, eye[...], (((0,), (0,)), ((), ())),
                         preferred_element_type=jnp.float32)
    out[...] = jnp.concatenate(
        [xt, jnp.zeros((_EBLK, 128 - _EDIM), jnp.float32)], axis=1)


def _tc_relayout(entT, eye):
    return pl.pallas_call(
        _relayout_body,
        grid=((_NENT + _EBLK - 1) // _EBLK,),
        in_specs=[
            pl.BlockSpec((_EDIM, _EBLK), lambda i: (0, i)),
            pl.BlockSpec((_EDIM, _EDIM), lambda i: (0, 0)),
        ],
        out_specs=pl.BlockSpec((_EBLK, 128), lambda i: (i, 0)),
        out_shape=jax.ShapeDtypeStruct((_NENT, 128), jnp.float32),
    )(entT, eye)


def _sc_gather_word(w_ids, word_emb, ntok):
    mesh = plsc.VectorSubcoreMesh(core_axis_name="c", subcore_axis_name="s")
    tpw = ntok // _NW
    half = tpw // 2

    @functools.partial(
        pl.kernel,
        mesh=mesh,
        out_type=jax.ShapeDtypeStruct((ntok, _HID), jnp.float32),
        scratch_types=[
            pltpu.VMEM((tpw,), jnp.int32),
            pltpu.VMEM((2, half // 2, _HID), jnp.float32),
            pltpu.VMEM((2, half // 2, _HID), jnp.float32),
            pltpu.SemaphoreType.DMA,
            pltpu.SemaphoreType.DMA,
            pltpu.SemaphoreType.DMA,
        ],
    )
    def gather(wids_h, wemb_h, W_h, widx_v, bufA, bufB, gsem, wsemA, wsemB):
        wid = lax.axis_index("s") * 2 + lax.axis_index("c")
        base = wid * tpw
        q = tpw // 4
        pltpu.sync_copy(wids_h.at[pl.ds(base, tpw)], widx_v)
        bufs = (bufA.at[0], bufA.at[1], bufB.at[0], bufB.at[1])
        wsems = (wsemA, wsemA, wsemB, wsemB)
        gs = [pltpu.async_copy(wemb_h.at[widx_v.at[pl.ds(i * q, q)]],
                               bufs[i], gsem) for i in range(4)]
        ws = []
        for i in range(4):
            gs[i].wait()
            ws.append(pltpu.async_copy(
                bufs[i], W_h.at[pl.ds(base + i * q, q)], wsems[i]))
        for w in ws:
            w.wait()

    return gather(w_ids, word_emb)


def _sc_gather_ent(k_ent, ent128, ntok):
    mesh = plsc.VectorSubcoreMesh(core_axis_name="c", subcore_axis_name="s")
    tpw = ntok // _NW

    @functools.partial(
        pl.kernel,
        mesh=mesh,
        out_type=jax.ShapeDtypeStruct((ntok, 128), jnp.float32),
        scratch_types=[
            pltpu.VMEM((tpw,), jnp.int32),
            pltpu.VMEM((tpw, 128), jnp.float32),
            pltpu.SemaphoreType.DMA,
        ],
    )
    def gather(kent_h, ent_h, E_h, keidx_v, ebuf, sem):
        wid = lax.axis_index("s") * 2 + lax.axis_index("c")
        base = wid * tpw
        pltpu.sync_copy(kent_h.at[pl.ds(base, tpw)], keidx_v)
        pltpu.async_copy(ent_h.at[keidx_v], ebuf, sem).wait()
        pltpu.sync_copy(ebuf, E_h.at[pl.ds(base, tpw)])

    return gather(k_ent, ent128)


def _tc_body(W, Es, wtt, ktt, wpos, kpos, kmask,
             ptab, wg, wb, keW, keb, kg, kb, out):
    cols = lax.broadcasted_iota(jnp.int32, (_S, _PT), 1)
    pt = ptab[...]
    oh_w = ((cols == wpos[0]) | (cols == wtt[0] + _MAXP)).astype(jnp.float32)
    wsum = W[0] + lax.dot_general(oh_w, pt, (((1,), (0,)), ((), ())),
                                  preferred_element_type=jnp.float32)
    u = jnp.mean(wsum, axis=-1, keepdims=True)
    d = wsum - u
    s = jnp.mean(d * d, axis=-1, keepdims=True)
    wemb = wg[...] * d / jnp.sqrt(s + _EPS) + wb[...]
    km = kmask[0]
    proj = lax.dot_general(Es[0][:, 0:_EDIM], keW[...], (((1,), (1,)), ((), ())),
                           preferred_element_type=jnp.float32)
    oh_k = ((cols == kpos[0]) | (cols == ktt[0] + _MAXP)).astype(jnp.float32)
    ptk = lax.dot_general(oh_k, pt, (((1,), (0,)), ((), ())),
                          preferred_element_type=jnp.float32)
    ksum = (proj + keb[...] + ptk) * km
    uk = jnp.mean(ksum, axis=-1, keepdims=True)
    dk = ksum - uk
    sk = jnp.mean(dk * dk, axis=-1, keepdims=True)
    kemb = kg[...] * dk / jnp.sqrt(sk + _EPS) + kb[...]
    out[0, 0:_S, :] = wemb
    out[0, _S:2 * _S, :] = kemb


def _tc_dense(W, Es, wtt, ktt, wpos, kpos, kmask, ptab, wg, wb, keW, keb, kg, kb):
    nb = W.shape[0]
    b3 = lambda i: (i, 0, 0)
    b2 = lambda i: (0, 0)
    return pl.pallas_call(
        _tc_body,
        grid=(nb,),
        in_specs=[
            pl.BlockSpec((1, _S, _HID), b3),
            pl.BlockSpec((1, _S, 128), b3),
            pl.BlockSpec((1, _S, 1), b3),
            pl.BlockSpec((1, _S, 1), b3),
            pl.BlockSpec((1, _S, 1), b3),
            pl.BlockSpec((1, _S, 1), b3),
            pl.BlockSpec((1, _S, 1), b3),
            pl.BlockSpec((_PT, _HID), b2),
            pl.BlockSpec((1, _HID), b2),
            pl.BlockSpec((1, _HID), b2),
            pl.BlockSpec((_HID, _EDIM), b2),
            pl.BlockSpec((1, _HID), b2),
            pl.BlockSpec((1, _HID), b2),
            pl.BlockSpec((1, _HID), b2),
        ],
        out_specs=pl.BlockSpec((1, 2 * _S, _HID), b3),
        out_shape=jax.ShapeDtypeStruct((nb, 2 * _S, _HID), jnp.float32),
    )(W, Es, wtt, ktt, wpos, kpos, kmask, ptab, wg, wb, keW, keb, kg, kb)


def kernel(input_ids, token_type_ids, word_emb, pos_emb, tt_emb, wln_g, wln_b,
           ke_W, ke_b, kln_g, kln_b, entityVec):
    ids_f = input_ids.astype(jnp.int32).reshape(_NTOK)
    tts_f = token_type_ids.astype(jnp.int32).reshape(_NTOK)
    w_ids, w_tt, w_pos, k_ent, k_tt, k_pos, k_mask = _sc_split(ids_f, tts_f)

    entT = jnp.transpose(entityVec)
    eye = jnp.eye(_EDIM, dtype=jnp.float32)
    ent128 = _tc_relayout(entT, eye)

    ptab = jnp.concatenate([pos_emb, tt_emb], axis=0)

    W = _sc_gather_word(w_ids, word_emb, _NTOK)
    k_ent_t, _ = lax.optimization_barrier((k_ent, W[0, 0]))
    Es = _sc_gather_ent(k_ent_t, ent128, _NTOK)

    out = _tc_dense(
        W.reshape(_B, _S, _HID), Es.reshape(_B, _S, 128),
        w_tt.reshape(_B, _S, 1), k_tt.reshape(_B, _S, 1),
        w_pos.reshape(_B, _S, 1), k_pos.reshape(_B, _S, 1),
        k_mask.reshape(_B, _S, 1),
        ptab, wln_g.reshape(1, _HID), wln_b.reshape(1, _HID),
        ke_W, ke_b.reshape(1, _HID), kln_g.reshape(1, _HID),
        kln_b.reshape(1, _HID))
    return out


# speedup vs baseline: 1.0618x; 1.0618x over previous
import functools

import jax
import jax.numpy as jnp
from jax import lax
from jax.experimental import pallas as pl
from jax.experimental.pallas import tpu as pltpu
from jax.experimental.pallas import tpu_sc as plsc

_VOCAB = 30522
_NENT = 100000
_EDIM = 100
_HID = 768
_MAXP = 512
_B = 16
_S = 256
_NTOK = _B * _S
_NW = 32
_TPW = _NTOK // _NW
_EPS = 1e-12
_PT = _MAXP + 2


def _splat_last(x, L=16):
    idx = jnp.full((L, 1), L - 1, jnp.int32)
    dn = lax.GatherDimensionNumbers(offset_dims=(), collapsed_slice_dims=(0,),
                                    start_index_map=(0,))
    return lax.gather(x, idx, dn, (1,),
                      mode=lax.GatherScatterMode.PROMISE_IN_BOUNDS)


def _sc_split_gather(ids_f, tts_f, word_emb):
    mesh = plsc.VectorSubcoreMesh(core_axis_name="c", subcore_axis_name="s")
    L = 16
    nchunk = _S // L
    NC = 8
    CR = _S // NC

    @functools.partial(
        pl.kernel,
        mesh=mesh,
        compiler_params=pltpu.CompilerParams(needs_layout_passes=False),
        out_type=[jax.ShapeDtypeStruct((_NTOK, _HID), jnp.float32)]
                 + [jax.ShapeDtypeStruct((_NTOK,), jnp.int32)] * 6
                 + [jax.ShapeDtypeStruct((_NTOK,), jnp.float32)],
        scratch_types=[
            pltpu.VMEM((_S,), jnp.int32),
            pltpu.VMEM((_S,), jnp.int32),
            pltpu.VMEM((_S,), jnp.int32),
            pltpu.VMEM((_S,), jnp.int32),
            pltpu.VMEM((_S,), jnp.int32),
            pltpu.VMEM((_S,), jnp.int32),
            pltpu.VMEM((_S,), jnp.int32),
            pltpu.VMEM((_S,), jnp.int32),
            pltpu.VMEM((_S,), jnp.float32),
            pltpu.VMEM((4, CR, _HID), jnp.float32),
            pltpu.SemaphoreType.DMA,
            pltpu.SemaphoreType.DMA,
            pltpu.SemaphoreType.DMA,
            pltpu.SemaphoreType.DMA,
        ],
    )
    def split(ids_h, tts_h, wemb_h,
              W_h, wid_o, wtt_o, wpos_o, kent_o, ktt_o, kpos_o, kmsk_o,
              idsv, ttsv, wiv, wtv, wpv, kev, ktv, kpv, kmv, gbuf,
              gsem, ws0, ws1, ws2):
        wid = lax.axis_index("s") * 2 + lax.axis_index("c")

        @pl.when(wid < _B)
        def _():
            base = wid * _S
            pltpu.sync_copy(ids_h.at[pl.ds(base, _S)], idsv)
            pltpu.sync_copy(tts_h.at[pl.ds(base, _S)], ttsv)
            nwv = jnp.zeros((L,), jnp.int32)
            nkv = jnp.zeros((L,), jnp.int32)
            for c in range(nchunk):
                v = idsv[pl.ds(c * L, L)]
                t = ttsv[pl.ds(c * L, L)]
                colv = lax.iota(jnp.int32, L) + (c * L)
                wm = (v > 0) & (v < _VOCAB)
                wmi = jnp.where(wm, jnp.int32(1), jnp.int32(0))
                wcum = plsc.cumsum(wmi)
                wdest = jnp.where(wm, nwv + wcum - 1, 0)
                plsc.store_scatter(wiv, [wdest], v, mask=wm)
                plsc.store_scatter(wtv, [wdest], t, mask=wm)
                plsc.store_scatter(wpv, [wdest], colv, mask=wm)
                nwv = nwv + _splat_last(wcum)
                km = v >= _VOCAB
                kmi = jnp.where(km, jnp.int32(1), jnp.int32(0))
                kcum = plsc.cumsum(kmi)
                kdest = jnp.where(km, nkv + kcum - 1, 0)
                plsc.store_scatter(kev, [kdest], v - _VOCAB, mask=km)
                plsc.store_scatter(ktv, [kdest], t, mask=km)
                plsc.store_scatter(kpv, [kdest], colv, mask=km)
                nkv = nkv + _splat_last(kcum)
            nkeff = jnp.where(nkv >= 2, nkv, 0)
            for c in range(nchunk):
                sl = pl.ds(c * L, L)
                colv = lax.iota(jnp.int32, L) + (c * L)
                wvld = colv < nwv
                wiv[sl] = jnp.where(wvld, wiv[sl], 0)
                wtv[sl] = jnp.where(wvld, wtv[sl], 1)
                wpv[sl] = jnp.where(wvld, wpv[sl], colv)
                kvld = colv < nkeff
                kev[sl] = jnp.where(kvld, kev[sl], 0)
                ktv[sl] = jnp.where(kvld, ktv[sl], 0)
                kpv[sl] = jnp.where(kvld, kpv[sl], 0)
                kmv[sl] = jnp.where(kvld, 1.0, 0.0)
            wsems = (ws0, ws1, ws2, ws0)
            gs = [None] * NC
            ws = [None] * NC
            for i in range(4):
                gs[i] = pltpu.async_copy(
                    wemb_h.at[wiv.at[pl.ds(i * CR, CR)]], gbuf.at[i], gsem)
            for i in range(NC):
                gs[i].wait()
                ws[i] = pltpu.async_copy(
                    gbuf.at[i % 4], W_h.at[pl.ds(base + i * CR, CR)],
                    wsems[i % 4])
                if i + 4 < NC:
                    ws[i].wait()
                    gs[i + 4] = pltpu.async_copy(
                        wemb_h.at[wiv.at[pl.ds((i + 4) * CR, CR)]],
                        gbuf.at[i % 4], gsem)
                    ws[i] = None
            pltpu.sync_copy(wiv, wid_o.at[pl.ds(base, _S)])
            pltpu.sync_copy(wtv, wtt_o.at[pl.ds(base, _S)])
            pltpu.sync_copy(wpv, wpos_o.at[pl.ds(base, _S)])
            pltpu.sync_copy(kev, kent_o.at[pl.ds(base, _S)])
            pltpu.sync_copy(ktv, ktt_o.at[pl.ds(base, _S)])
            pltpu.sync_copy(kpv, kpos_o.at[pl.ds(base, _S)])
            pltpu.sync_copy(kmv, kmsk_o.at[pl.ds(base, _S)])
            for w in ws:
                if w is not None:
                    w.wait()

    return split(ids_f, tts_f, word_emb)


_EBLK = 2048


def _relayout_body(entT, eye, out):
    x = entT[...]
    xt = lax.dot_general(x, eye[...], (((0,), (0,)), ((), ())),
                         preferred_element_type=jnp.float32)
    out[...] = jnp.concatenate(
        [xt, jnp.zeros((_EBLK, 128 - _EDIM), jnp.float32)], axis=1)


def _tc_relayout(entT, eye):
    return pl.pallas_call(
        _relayout_body,
        grid=((_NENT + _EBLK - 1) // _EBLK,),
        in_specs=[
            pl.BlockSpec((_EDIM, _EBLK), lambda i: (0, i)),
            pl.BlockSpec((_EDIM, _EDIM), lambda i: (0, 0)),
        ],
        out_specs=pl.BlockSpec((_EBLK, 128), lambda i: (i, 0)),
        out_shape=jax.ShapeDtypeStruct((_NENT, 128), jnp.float32),
    )(entT, eye)


def _sc_gather_ent(k_ent, ent128, ntok):
    mesh = plsc.VectorSubcoreMesh(core_axis_name="c", subcore_axis_name="s")
    tpw = ntok // _NW

    @functools.partial(
        pl.kernel,
        mesh=mesh,
        out_type=jax.ShapeDtypeStruct((ntok, 128), jnp.float32),
        scratch_types=[
            pltpu.VMEM((tpw,), jnp.int32),
            pltpu.VMEM((tpw, 128), jnp.float32),
            pltpu.SemaphoreType.DMA,
        ],
    )
    def gather(kent_h, ent_h, E_h, keidx_v, ebuf, sem):
        wid = lax.axis_index("s") * 2 + lax.axis_index("c")
        base = wid * tpw
        pltpu.sync_copy(kent_h.at[pl.ds(base, tpw)], keidx_v)
        pltpu.async_copy(ent_h.at[keidx_v], ebuf, sem).wait()
        pltpu.sync_copy(ebuf, E_h.at[pl.ds(base, tpw)])

    return gather(k_ent, ent128)


def _tc_body(W, Es, wtt, ktt, wpos, kpos, kmask,
             ptab, wg, wb, keW, keb, kg, kb, out):
    cols = lax.broadcasted_iota(jnp.int32, (_S, _PT), 1)
    pt = ptab[...]
    oh_w = ((cols == wpos[0]) | (cols == wtt[0] + _MAXP)).astype(jnp.float32)
    wsum = W[0] + lax.dot_general(oh_w, pt, (((1,), (0,)), ((), ())),
                                  preferred_element_type=jnp.float32)
    u = jnp.mean(wsum, axis=-1, keepdims=True)
    d = wsum - u
    s = jnp.mean(d * d, axis=-1, keepdims=True)
    wemb = wg[...] * d / jnp.sqrt(s + _EPS) + wb[...]
    km = kmask[0]
    proj = lax.dot_general(Es[0][:, 0:_EDIM], keW[...], (((1,), (1,)), ((), ())),
                           preferred_element_type=jnp.float32)
    oh_k = ((cols == kpos[0]) | (cols == ktt[0] + _MAXP)).astype(jnp.float32)
    ptk = lax.dot_general(oh_k, pt, (((1,), (0,)), ((), ())),
                          preferred_element_type=jnp.float32)
    ksum = (proj + keb[...] + ptk) * km
    uk = jnp.mean(ksum, axis=-1, keepdims=True)
    dk = ksum - uk
    sk = jnp.mean(dk * dk, axis=-1, keepdims=True)
    kemb = kg[...] * dk / jnp.sqrt(sk + _EPS) + kb[...]
    out[0, 0:_S, :] = wemb
    out[0, _S:2 * _S, :] = kemb


def _tc_dense(W, Es, wtt, ktt, wpos, kpos, kmask, ptab, wg, wb, keW, keb, kg, kb):
    nb = W.shape[0]
    b3 = lambda i: (i, 0, 0)
    b2 = lambda i: (0, 0)
    return pl.pallas_call(
        _tc_body,
        grid=(nb,),
        in_specs=[
            pl.BlockSpec((1, _S, _HID), b3),
            pl.BlockSpec((1, _S, 128), b3),
            pl.BlockSpec((1, _S, 1), b3),
            pl.BlockSpec((1, _S, 1), b3),
            pl.BlockSpec((1, _S, 1), b3),
            pl.BlockSpec((1, _S, 1), b3),
            pl.BlockSpec((1, _S, 1), b3),
            pl.BlockSpec((_PT, _HID), b2),
            pl.BlockSpec((1, _HID), b2),
            pl.BlockSpec((1, _HID), b2),
            pl.BlockSpec((_HID, _EDIM), b2),
            pl.BlockSpec((1, _HID), b2),
            pl.BlockSpec((1, _HID), b2),
            pl.BlockSpec((1, _HID), b2),
        ],
        out_specs=pl.BlockSpec((1, 2 * _S, _HID), b3),
        out_shape=jax.ShapeDtypeStruct((nb, 2 * _S, _HID), jnp.float32),
    )(W, Es, wtt, ktt, wpos, kpos, kmask, ptab, wg, wb, keW, keb, kg, kb)


def kernel(input_ids, token_type_ids, word_emb, pos_emb, tt_emb, wln_g, wln_b,
           ke_W, ke_b, kln_g, kln_b, entityVec):
    ids_f = input_ids.astype(jnp.int32).reshape(_NTOK)
    tts_f = token_type_ids.astype(jnp.int32).reshape(_NTOK)
    W, w_ids, w_tt, w_pos, k_ent, k_tt, k_pos, k_mask = _sc_split_gather(
        ids_f, tts_f, word_emb)

    entT = jnp.transpose(entityVec)
    eye = jnp.eye(_EDIM, dtype=jnp.float32)
    ent128 = _tc_relayout(entT, eye)

    ptab = jnp.concatenate([pos_emb, tt_emb], axis=0)

    Es = _sc_gather_ent(k_ent, ent128, _NTOK)

    out = _tc_dense(
        W.reshape(_B, _S, _HID), Es.reshape(_B, _S, 128),
        w_tt.reshape(_B, _S, 1), k_tt.reshape(_B, _S, 1),
        w_pos.reshape(_B, _S, 1), k_pos.reshape(_B, _S, 1),
        k_mask.reshape(_B, _S, 1),
        ptab, wln_g.reshape(1, _HID), wln_b.reshape(1, _HID),
        ke_W, ke_b.reshape(1, _HID), kln_g.reshape(1, _HID),
        kln_b.reshape(1, _HID))
    return out
